# Initial kernel scaffold; baseline (speedup 1.0000x reference)
import jax
import jax.numpy as jnp
from jax.experimental import pallas as pl


def kernel(Corpus_, batch_inputs, entity_embeddings, relation_embed, edge_list, edge_type, edge_embed, edge_list_nhop, edge_type_nhop, a_head0, a2_head0, a_head1, a2_head1, a_out, a2_out, W):
    raise NotImplementedError("write your pallas kernel here")


# speedup vs baseline: 2.7444x; 2.7444x over previous
import functools

import jax
import jax.numpy as jnp
from jax import lax
from jax.experimental import pallas as pl
from jax.experimental.pallas import tpu as pltpu
from jax.experimental.pallas import tpu_sc as plsc

N_NODES = 10000
SN = 10112
NFEAT = 128
NHID = 32
RELDIM = 64
NRELA = 500
RP = 512
ALPHA = 0.2

E1 = 160000
E1P = 163840
E1W = E1P // 32
E2 = 50000
E2P = 53248
E2W = E2P // 32
C = 64
ROWW = 72
NSTRIPES = SN // C

_f32 = jnp.float32
_i32 = jnp.int32


def _tc_a1(x0p, relp, ah0, ah1, a2h0, a2h1, aout, a2out, W,
           usrc, udst, snode1, t3q, rfull, r2q, srel, ortho):
    A1 = jnp.concatenate([ah0[:, :NFEAT], ah1[:, :NFEAT]], axis=0)
    A2 = jnp.concatenate([ah0[:, NFEAT:2 * NFEAT], ah1[:, NFEAT:2 * NFEAT]], axis=0)
    A3 = jnp.concatenate([ah0[:, 2 * NFEAT:], ah1[:, 2 * NFEAT:]], axis=0)

    dn = (((1,), (1,)), ((), ()))
    Us = lax.dot_general(x0p[...], A1, dn)
    Ud = lax.dot_general(x0p[...], A2, dn)
    usrc[...] = Us
    udst[...] = Ud

    snode1[0:1, :] = lax.dot_general(a2h0[...], Us[:, :NHID], dn)
    snode1[1:2, :] = lax.dot_general(a2h1[...], Us[:, NHID:], dn)
    snode1[2:3, :] = lax.dot_general(a2h0[...], Ud[:, :NHID], dn)
    snode1[3:4, :] = lax.dot_general(a2h1[...], Ud[:, NHID:], dn)

    T3 = lax.dot_general(relp[...], A3, dn)
    t3q[...] = T3
    R = lax.dot_general(relp[...], W[...], (((1,), (0,)), ((), ())))
    rfull[...] = R
    B3 = aout[:, 2 * RELDIM:]
    R2 = lax.dot_general(R, B3, dn)
    r2q[...] = R2

    srel[0:1, :] = lax.dot_general(a2h0[...], T3[:, :NHID], dn)
    srel[1:2, :] = lax.dot_general(a2h1[...], T3[:, NHID:], dn)
    srel[2:3, :] = lax.dot_general(a2out[...], R2, dn)
    srel[3:4, :] = jnp.zeros((1, RP), _f32)

    tot = jnp.float32(0.0)
    for a in (ah0[...], ah1[...], aout[...]):
        hd = a.shape[0] // 2
        ahh = a.reshape(2, hd, a.shape[1])
        gram = lax.dot_general(ahh, ahh, (((2,), (2,)), ((0,), (0,))))
        ii = lax.broadcasted_iota(_i32, (hd, hd), 0)
        jj = lax.broadcasted_iota(_i32, (hd, hd), 1)
        eye = jnp.where(ii == jj, jnp.float32(1.0), jnp.float32(0.0))
        tot = tot + 0.01 * jnp.sum((gram - eye[None]) ** 2)
    ortho[...] = jnp.reshape(tot, (1, 1))


def _tc_a2(eeb, a2h0, a2h1, ah0, ah1, gq, sg):
    A3 = jnp.concatenate([ah0[:, 2 * NFEAT:], ah1[:, 2 * NFEAT:]], axis=0)
    dn = (((1,), (1,)), ((), ()))
    G = lax.dot_general(eeb[...], A3, dn)
    gq[...] = G
    sg[0:1, :] = lax.dot_general(a2h0[...], G[:, :NHID], dn)
    sg[1:2, :] = lax.dot_general(a2h1[...], G[:, NHID:], dn)


def _tc_b(acc, aout, a2out, vsrc, vdst, snode2):
    s = acc[0] + acc[1]
    w0 = s[:, 64:65]
    w1 = s[:, 65:66]
    w0 = jnp.where(w0 == 0.0, jnp.float32(1e-12), w0)
    w1 = jnp.where(w1 == 0.0, jnp.float32(1e-12), w1)
    h0 = s[:, :NHID] / w0
    h1 = s[:, NHID:2 * NHID] / w1
    x1 = jnp.concatenate([_elu(h0), _elu(h1)], axis=1)
    dn = (((1,), (1,)), ((), ()))
    B1 = aout[:, :RELDIM]
    B2 = aout[:, RELDIM:2 * RELDIM]
    Vs = lax.dot_general(x1, B1, dn)
    Vd = lax.dot_general(x1, B2, dn)
    vsrc[...] = Vs
    vdst[...] = Vd
    snode2[0:1, :] = lax.dot_general(a2out[...], Vs, dn)
    snode2[1:2, :] = lax.dot_general(a2out[...], Vd, dn)
    snode2[2:4, :] = jnp.zeros((2, SN), _f32)


def _elu(x):
    return jnp.where(x > 0, x, jnp.exp(jnp.minimum(x, 0.0)) - 1.0)


def _tc_c(acc, out):
    s = acc[0] + acc[1]
    w = s[:, 64:65]
    w = jnp.where(w == 0.0, jnp.float32(1e-12), w)
    out[...] = (s[:, :RELDIM] / w)[:N_NODES, :]


LINEAR, G1, G2 = 0, 1, 2


def _sc_chunk(mode, nheads, base, us, ud, src_h, dst_h, gq_h, sg_h, t0_h, t1_h,
              tq_h, srcv, dstv, t0v, t1v, rowsA, rowsB, qa, qb, sqv, outr, wb,
              ssrcv, sdstv, srelv, accsh, sem):
    seg = 64 // nheads
    pltpu.sync_copy(src_h.at[pl.ds(base, C)], srcv)
    pltpu.sync_copy(dst_h.at[pl.ds(base, C)], dstv)
    if mode == LINEAR:
        pltpu.sync_copy(gq_h.at[pl.ds(base, C)], qa)
        for h in range(nheads):
            pltpu.sync_copy(sg_h[h].at[pl.ds(base, C)], sqv[h])
    elif mode == G1:
        pltpu.sync_copy(t0_h.at[pl.ds(base, C)], t0v)
        pltpu.async_copy(tq_h.at[t0v], qa, sem).wait()
    else:
        pltpu.sync_copy(t0_h.at[pl.ds(base, C)], t0v)
        pltpu.sync_copy(t1_h.at[pl.ds(base, C)], t1v)
        da = pltpu.async_copy(tq_h.at[t0v], qa, sem)
        db = pltpu.async_copy(tq_h.at[t1v], qb, sem)
        da.wait()
        db.wait()
    ga = pltpu.async_copy(us.at[srcv], rowsA, sem)
    gb = pltpu.async_copy(ud.at[dstv], rowsB, sem)
    ga.wait()
    gb.wait()

    lanes = lax.broadcasted_iota(_i32, (16,), 0)
    for g in range(C // 16):
        s16 = srcv[pl.ds(g * 16, 16)]
        d16 = dstv[pl.ds(g * 16, 16)]
        if mode != LINEAR:
            t016 = t0v[pl.ds(g * 16, 16)]
        if mode == G2:
            t116 = t1v[pl.ds(g * 16, 16)]
        for h in range(nheads):
            sc = plsc.load_gather(ssrcv[h], [s16]) + plsc.load_gather(sdstv[h], [d16])
            if mode == LINEAR:
                sc = sc + sqv[h][pl.ds(g * 16, 16)]
            elif mode == G1:
                sc = sc + plsc.load_gather(srelv[h], [t016])
            else:
                sc = sc + plsc.load_gather(srelv[h], [t016]) + plsc.load_gather(srelv[h], [t116])
            w = jnp.exp(jnp.where(sc > 0, -sc, (-ALPHA) * sc))
            wb[h][pl.ds(g * 16, 16)] = w
            plsc.store_scatter(outr, [g * 16 + lanes, jnp.full((16,), 64 + h, _i32)], w)

    def edge_body(i, _):
        for h in range(nheads):
            wv = plsc.load_gather(wb[h], [jnp.full((16,), i, _i32)])
            for jj in range(seg // 16):
                j = h * (seg // 16) + jj
                m = rowsA[i, pl.ds(j * 16, 16)] + rowsB[i, pl.ds(j * 16, 16)] \
                    + qa[i, pl.ds(j * 16, 16)]
                if mode == G2:
                    m = m + qb[i, pl.ds(j * 16, 16)]
                outr[i, pl.ds(j * 16, 16)] = wv * m
        return 0

    lax.fori_loop(0, C, edge_body, 0)
    pltpu.sync_copy(outr, accsh.at[srcv], add=True)


def _make_sc_kernel(nheads, qmode_a):
    seg = 64 // nheads

    def body(*refs):
        if qmode_a == LINEAR:
            (srcA, dstA, gq, sg0, sg1, srcB, dstB, t0B, t1B, tq, srel, us, ud,
             snode, acc) = refs[:15]
            scr = refs[15:]
            sgl = [sg0, sg1]
            tyA = None
        else:
            (srcA, dstA, tyA, srcB, dstB, t0B, t1B, tq, srel, us, ud,
             snode, acc) = refs[:13]
            scr = refs[13:]
            sgl = [None, None]
            gq = None
        (srcv, dstv, t0v, t1v, rowsA, rowsB, qa, qb, outr) = scr[:9]
        scr = scr[9:]
        sqv = list(scr[:nheads]); scr = scr[nheads:]
        wb = list(scr[:nheads]); scr = scr[nheads:]
        ssrcv = list(scr[:nheads]); scr = scr[nheads:]
        sdstv = list(scr[:nheads]); scr = scr[nheads:]
        srelv = list(scr[:nheads]); scr = scr[nheads:]
        accsh, sem = scr

        cid = lax.axis_index("c")
        sid = lax.axis_index("s")
        wid = cid * 16 + sid

        z16 = jnp.zeros((16,), _f32)

        def zero_body(i, _):
            for j in range(4):
                outr[i, pl.ds(j * 16, 16)] = z16
            outr[i, pl.ds(ROWW - 16, 16)] = z16
            return 0

        lax.fori_loop(0, C, zero_body, 0)

        for h in range(nheads):
            pltpu.sync_copy(snode.at[h], ssrcv[h])
            pltpu.sync_copy(snode.at[nheads + h], sdstv[h])
            pltpu.sync_copy(srel.at[h], srelv[h])

        for k in range((NSTRIPES + 15) // 16):
            stripe = sid + k * 16

            @pl.when(stripe < NSTRIPES)
            def _():
                pltpu.sync_copy(outr, accsh.at[pl.ds(stripe * C, C)])

        plsc.subcore_barrier()

        def chunk_a(k, _):
            base = pl.multiple_of(wid * E1W + k * C, C)
            _sc_chunk(qmode_a, nheads, base, us, ud, srcA, dstA, gq, sgl, tyA,
                      None, tq, srcv, dstv, t0v, t1v, rowsA, rowsB, qa, qb, sqv,
                      outr, wb, ssrcv, sdstv, srelv, accsh, sem)
            return 0

        def chunk_b(k, _):
            base = pl.multiple_of(wid * E2W + k * C, C)
            _sc_chunk(G2, nheads, base, us, ud, srcB, dstB, None, [None, None],
                      t0B, t1B, tq, srcv, dstv, t0v, t1v, rowsA, rowsB, qa, qb,
                      sqv, outr, wb, ssrcv, sdstv, srelv, accsh, sem)
            return 0

        lax.fori_loop(0, E1W // C, chunk_a, 0)
        lax.fori_loop(0, E2W // C, chunk_b, 0)

        plsc.subcore_barrier()

        for k in range((NSTRIPES + 15) // 16):
            stripe = sid + k * 16

            @pl.when(stripe < NSTRIPES)
            def _():
                pltpu.sync_copy(accsh.at[pl.ds(stripe * C, C)],
                                acc.at[cid, pl.ds(stripe * C, C)])

    scratch = [
        pltpu.VMEM((C,), _i32), pltpu.VMEM((C,), _i32),
        pltpu.VMEM((C,), _i32), pltpu.VMEM((C,), _i32),
        pltpu.VMEM((C, 64), _f32), pltpu.VMEM((C, 64), _f32),
        pltpu.VMEM((C, 64), _f32), pltpu.VMEM((C, 64), _f32),
        pltpu.VMEM((C, ROWW), _f32),
    ]
    scratch += [pltpu.VMEM((C,), _f32)] * nheads
    scratch += [pltpu.VMEM((C,), _f32)] * nheads
    scratch += [pltpu.VMEM((SN,), _f32)] * nheads
    scratch += [pltpu.VMEM((SN,), _f32)] * nheads
    scratch += [pltpu.VMEM((RP,), _f32)] * nheads
    scratch += [pltpu.VMEM_SHARED((SN, ROWW), _f32), pltpu.SemaphoreType.DMA]

    mesh = plsc.VectorSubcoreMesh(core_axis_name="c", subcore_axis_name="s",
                                  num_cores=2, num_subcores=16)
    return pl.kernel(
        body,
        out_type=jax.ShapeDtypeStruct((2, SN, ROWW), _f32),
        mesh=mesh,
        scratch_types=scratch,
        compiler_params=pltpu.CompilerParams(needs_layout_passes=False,
                                             use_tc_tiling_on_sc=False),
    )


def kernel(Corpus_, batch_inputs, entity_embeddings, relation_embed, edge_list,
           edge_type, edge_embed, edge_list_nhop, edge_type_nhop, a_head0,
           a2_head0, a_head1, a2_head1, a_out, a2_out, W):
    x0p = jnp.pad(entity_embeddings, ((0, SN - N_NODES), (0, 0)))
    relp = jnp.pad(relation_embed, ((0, RP - NRELA), (0, 0)))
    eep = jnp.pad(edge_embed, ((0, E1P - E1), (0, 0)))

    srcA = jnp.concatenate([edge_list[0], jnp.full((E1P - E1,), N_NODES, _i32)]).astype(_i32)
    dstA = jnp.concatenate([edge_list[1], jnp.zeros((E1P - E1,), _i32)]).astype(_i32)
    tyA = jnp.concatenate([edge_type, jnp.zeros((E1P - E1,), _i32)]).astype(_i32)
    srcB = jnp.concatenate([edge_list_nhop[0], jnp.full((E2P - E2,), N_NODES, _i32)]).astype(_i32)
    dstB = jnp.concatenate([edge_list_nhop[1], jnp.zeros((E2P - E2,), _i32)]).astype(_i32)
    t0B = jnp.concatenate([edge_type_nhop[:, 0], jnp.zeros((E2P - E2,), _i32)]).astype(_i32)
    t1B = jnp.concatenate([edge_type_nhop[:, 1], jnp.zeros((E2P - E2,), _i32)]).astype(_i32)

    usrc, udst, snode1, t3q, rfull, r2q, srel, ortho = pl.pallas_call(
        _tc_a1,
        out_shape=[
            jax.ShapeDtypeStruct((SN, 64), _f32),
            jax.ShapeDtypeStruct((SN, 64), _f32),
            jax.ShapeDtypeStruct((4, SN), _f32),
            jax.ShapeDtypeStruct((RP, 64), _f32),
            jax.ShapeDtypeStruct((RP, 64), _f32),
            jax.ShapeDtypeStruct((RP, 64), _f32),
            jax.ShapeDtypeStruct((4, RP), _f32),
            jax.ShapeDtypeStruct((1, 1), _f32),
        ],
    )(x0p, relp, a_head0, a_head1, a2_head0, a2_head1, a_out, a2_out, W)

    BLK = 2048
    nblk = E1P // BLK
    gq1, sg1 = pl.pallas_call(
        _tc_a2,
        grid=(nblk,),
        in_specs=[
            pl.BlockSpec((BLK, RELDIM), lambda i: (i, 0)),
            pl.BlockSpec((1, NHID), lambda i: (0, 0)),
            pl.BlockSpec((1, NHID), lambda i: (0, 0)),
            pl.BlockSpec((NHID, 2 * NFEAT + RELDIM), lambda i: (0, 0)),
            pl.BlockSpec((NHID, 2 * NFEAT + RELDIM), lambda i: (0, 0)),
        ],
        out_specs=[
            pl.BlockSpec((BLK, RELDIM), lambda i: (i, 0)),
            pl.BlockSpec((2, BLK), lambda i: (0, i)),
        ],
        out_shape=[
            jax.ShapeDtypeStruct((E1P, RELDIM), _f32),
            jax.ShapeDtypeStruct((2, E1P), _f32),
        ],
    )(eep, a2_head0, a2_head1, a_head0, a_head1)

    sc1 = _make_sc_kernel(2, LINEAR)
    acc1 = sc1(srcA, dstA, gq1, sg1[0], sg1[1], srcB, dstB, t0B, t1B,
               t3q, srel[0:2], usrc, udst, snode1)

    vsrc, vdst, snode2 = pl.pallas_call(
        _tc_b,
        out_shape=[
            jax.ShapeDtypeStruct((SN, 64), _f32),
            jax.ShapeDtypeStruct((SN, 64), _f32),
            jax.ShapeDtypeStruct((4, SN), _f32),
        ],
    )(acc1, a_out, a2_out)

    sc2 = _make_sc_kernel(1, G1)
    acc2 = sc2(srcA, dstA, tyA, srcB, dstB, t0B, t1B,
               r2q, srel[2:4], vsrc, vdst, snode2)

    x = pl.pallas_call(
        _tc_c,
        out_shape=jax.ShapeDtypeStruct((N_NODES, 64), _f32),
    )(acc2)

    return (x, rfull[:NRELA], ortho[0, 0])


# speedup vs baseline: 4.1747x; 1.5212x over previous
import jax
import jax.numpy as jnp
from jax import lax
from jax.experimental import pallas as pl
from jax.experimental.pallas import tpu as pltpu
from jax.experimental.pallas import tpu_sc as plsc

N_NODES = 10000
SN = 10112
NFEAT = 128
NHID = 32
RELDIM = 64
NRELA = 500
RP = 512
ALPHA = 0.2

E1 = 160000
E1P = 163840
E1W = E1P // 32
E2 = 50000
E2P = 53248
E2W = E2P // 32
C = 128
ROWW = 72
NSTRIPES = SN // C

_f32 = jnp.float32
_i32 = jnp.int32


def _tc_a1(x0p, relp, ah0, ah1, a2h0, a2h1, aout, a2out, W,
           usrc, udst, t3q, rfull, r2q, ortho):
    A1 = jnp.concatenate([ah0[:, :NFEAT], ah1[:, :NFEAT]], axis=0)
    A2 = jnp.concatenate([ah0[:, NFEAT:2 * NFEAT], ah1[:, NFEAT:2 * NFEAT]], axis=0)
    A3 = jnp.concatenate([ah0[:, 2 * NFEAT:], ah1[:, 2 * NFEAT:]], axis=0)

    dn = (((1,), (1,)), ((), ()))

    def table2(rows, out_ref):
        s0 = lax.dot_general(rows[:, :NHID], a2h0[...], (((1,), (1,)), ((), ())))
        s1 = lax.dot_general(rows[:, NHID:], a2h1[...], (((1,), (1,)), ((), ())))
        n = rows.shape[0]
        pad = jnp.zeros((n, ROWW - 66), _f32)
        out_ref[...] = jnp.concatenate([rows, s0, s1, pad], axis=1)

    Us = lax.dot_general(x0p[...], A1, dn)
    Ud = lax.dot_general(x0p[...], A2, dn)
    table2(Us, usrc)
    table2(Ud, udst)

    T3 = lax.dot_general(relp[...], A3, dn)
    table2(T3, t3q)

    R = lax.dot_general(relp[...], W[...], (((1,), (0,)), ((), ())))
    rfull[...] = R
    B3 = aout[:, 2 * RELDIM:]
    R2 = lax.dot_general(R, B3, dn)
    sO = lax.dot_general(R2, a2out[...], (((1,), (1,)), ((), ())))
    r2q[...] = jnp.concatenate(
        [R2, sO, jnp.zeros((RP, ROWW - 65), _f32)], axis=1)

    tot = jnp.float32(0.0)
    for a in (ah0[...], ah1[...], aout[...]):
        hd = a.shape[0] // 2
        ahh = a.reshape(2, hd, a.shape[1])
        gram = lax.dot_general(ahh, ahh, (((2,), (2,)), ((0,), (0,))))
        ii = lax.broadcasted_iota(_i32, (hd, hd), 0)
        jj = lax.broadcasted_iota(_i32, (hd, hd), 1)
        eye = jnp.where(ii == jj, jnp.float32(1.0), jnp.float32(0.0))
        tot = tot + 0.01 * jnp.sum((gram - eye[None]) ** 2)
    ortho[...] = jnp.reshape(tot, (1, 1))


def _tc_a2(eeb, a2h0, a2h1, ah0, ah1, gq):
    A3 = jnp.concatenate([ah0[:, 2 * NFEAT:], ah1[:, 2 * NFEAT:]], axis=0)
    dn = (((1,), (1,)), ((), ()))
    G = lax.dot_general(eeb[...], A3, dn)
    s0 = lax.dot_general(G[:, :NHID], a2h0[...], dn)
    s1 = lax.dot_general(G[:, NHID:], a2h1[...], dn)
    pad = jnp.zeros((G.shape[0], ROWW - 66), _f32)
    gq[...] = jnp.concatenate([G, s0, s1, pad], axis=1)


def _tc_b(acc, aout, a2out, vsrc, vdst):
    s = acc[0] + acc[1]
    w0 = s[:, 64:65]
    w1 = s[:, 65:66]
    w0 = jnp.where(w0 == 0.0, jnp.float32(1e-12), w0)
    w1 = jnp.where(w1 == 0.0, jnp.float32(1e-12), w1)
    h0 = s[:, :NHID] / w0
    h1 = s[:, NHID:2 * NHID] / w1
    x1 = jnp.concatenate([_elu(h0), _elu(h1)], axis=1)
    dn = (((1,), (1,)), ((), ()))
    B1 = aout[:, :RELDIM]
    B2 = aout[:, RELDIM:2 * RELDIM]
    pad = jnp.zeros((SN, ROWW - 65), _f32)
    Vs = lax.dot_general(x1, B1, dn)
    Vd = lax.dot_general(x1, B2, dn)
    vsrc[...] = jnp.concatenate(
        [Vs, lax.dot_general(Vs, a2out[...], dn), pad], axis=1)
    vdst[...] = jnp.concatenate(
        [Vd, lax.dot_general(Vd, a2out[...], dn), pad], axis=1)


def _elu(x):
    return jnp.where(x > 0, x, jnp.exp(jnp.minimum(x, 0.0)) - 1.0)


def _tc_c(acc, out):
    s = acc[0] + acc[1]
    w = s[:, 64:65]
    w = jnp.where(w == 0.0, jnp.float32(1e-12), w)
    out[...] = (s[:, :RELDIM] / w)[:N_NODES, :]


LINEAR, G1, G2 = 0, 1, 2


def _compute_chunk(mode, nheads, srcv, rowsA, rowsB, qa, qb, outr, wb, accsh):
    seg = 64 // nheads
    lanes = lax.broadcasted_iota(_i32, (16,), 0)
    for g in range(C // 16):
        e16 = g * 16 + lanes
        for h in range(nheads):
            c16 = jnp.full((16,), 64 + h, _i32)
            sc = plsc.load_gather(rowsA, [e16, c16]) \
                + plsc.load_gather(rowsB, [e16, c16]) \
                + plsc.load_gather(qa, [e16, c16])
            if mode == G2:
                sc = sc + plsc.load_gather(qb, [e16, c16])
            w = jnp.exp(jnp.where(sc > 0, -sc, (-ALPHA) * sc))
            wb[h][pl.ds(g * 16, 16)] = w
            plsc.store_scatter(outr, [e16, c16], w)

    def edge_body(i, _):
        for h in range(nheads):
            wv = plsc.load_gather(wb[h], [jnp.full((16,), i, _i32)])
            for jj in range(seg // 16):
                j = h * (seg // 16) + jj
                m = rowsA[i, pl.ds(j * 16, 16)] + rowsB[i, pl.ds(j * 16, 16)] \
                    + qa[i, pl.ds(j * 16, 16)]
                if mode == G2:
                    m = m + qb[i, pl.ds(j * 16, 16)]
                outr[i, pl.ds(j * 16, 16)] = wv * m
        return 0

    lax.fori_loop(0, C, edge_body, 0)
    pltpu.sync_copy(outr, accsh.at[srcv], add=True)


def _make_sc_kernel(nheads, qmode_a):
    def body(*refs):
        if qmode_a == LINEAR:
            (srcA, dstA, gq, srcB, dstB, t0B, t1B, tq, us, ud, acc) = refs[:11]
            scr = refs[11:]
            tyA = None
        else:
            (srcA, dstA, tyA, srcB, dstB, t0B, t1B, tq, us, ud, acc) = refs[:11]
            scr = refs[11:]
            gq = None
        (srcv0, dstv0, srcv1, dstv1, t0v, t1v,
         rowsA0, rowsB0, rowsA1, rowsB1, qa0, qa1, qb, outr) = scr[:14]
        scr = scr[14:]
        wb = list(scr[:nheads]); scr = scr[nheads:]
        accsh, sem0, sem1 = scr

        cid = lax.axis_index("c")
        sid = lax.axis_index("s")
        wid = cid * 16 + sid

        z16 = jnp.zeros((16,), _f32)

        def zero_body(i, _):
            for j in range(4):
                outr[i, pl.ds(j * 16, 16)] = z16
            outr[i, pl.ds(ROWW - 16, 16)] = z16
            return 0

        lax.fori_loop(0, C, zero_body, 0)

        for k in range((NSTRIPES + 15) // 16):
            stripe = sid + k * 16

            @pl.when(stripe < NSTRIPES)
            def _():
                pltpu.sync_copy(outr, accsh.at[pl.ds(stripe * C, C)])

        plsc.subcore_barrier()

        bufs = [
            (srcv0, dstv0, rowsA0, rowsB0, qa0, sem0),
            (srcv1, dstv1, rowsA1, rowsB1, qa1, sem1),
        ]

        def issue(mode, src_h, dst_h, gq_h, t0_h, wbase, k, b):
            srcv, dstv, rowsA, rowsB, qa, sem = bufs[b]
            base = pl.multiple_of(wbase + k * C, C)
            pltpu.sync_copy(src_h.at[pl.ds(base, C)], srcv)
            pltpu.sync_copy(dst_h.at[pl.ds(base, C)], dstv)
            pltpu.async_copy(us.at[srcv], rowsA, sem)
            pltpu.async_copy(ud.at[dstv], rowsB, sem)
            if mode == LINEAR:
                pltpu.async_copy(gq_h.at[pl.ds(base, C)], qa, sem)
            else:
                tv = t0v if b == 0 else t1v
                pltpu.sync_copy(t0_h.at[pl.ds(base, C)], tv)
                pltpu.async_copy(tq.at[tv], qa, sem)

        def wait_bufs(mode, b):
            srcv, dstv, rowsA, rowsB, qa, sem = bufs[b]
            pltpu.make_async_copy(us.at[srcv], rowsA, sem).wait()
            pltpu.make_async_copy(ud.at[dstv], rowsB, sem).wait()
            pltpu.make_async_copy(us.at[srcv], qa, sem).wait()

        def consume(mode, b):
            srcv, dstv, rowsA, rowsB, qa, sem = bufs[b]
            _compute_chunk(mode, nheads, srcv, rowsA, rowsB, qa, qb,
                           outr, wb, accsh)

        nch_a = E1W // C
        assert nch_a % 2 == 0
        ia = lambda k, b: issue(qmode_a, srcA, dstA, gq, tyA, wid * E1W, k, b)
        ia(0, 0)

        def pair_body(p, _):
            k = p * 2
            ia(k + 1, 1)
            wait_bufs(qmode_a, 0)
            consume(qmode_a, 0)

            @pl.when(k + 2 < nch_a)
            def _():
                ia(k + 2, 0)

            wait_bufs(qmode_a, 1)
            consume(qmode_a, 1)
            return 0

        lax.fori_loop(0, nch_a // 2, pair_body, 0)

        def chunk_b(k, _):
            base = pl.multiple_of(wid * E2W + k * C, C)
            pltpu.sync_copy(srcB.at[pl.ds(base, C)], srcv0)
            pltpu.sync_copy(dstB.at[pl.ds(base, C)], dstv0)
            pltpu.sync_copy(t0B.at[pl.ds(base, C)], t0v)
            pltpu.sync_copy(t1B.at[pl.ds(base, C)], t1v)
            pltpu.async_copy(us.at[srcv0], rowsA0, sem0)
            pltpu.async_copy(ud.at[dstv0], rowsB0, sem0)
            pltpu.async_copy(tq.at[t0v], qa0, sem0)
            pltpu.async_copy(tq.at[t1v], qb, sem0)
            pltpu.make_async_copy(us.at[srcv0], rowsA0, sem0).wait()
            pltpu.make_async_copy(ud.at[dstv0], rowsB0, sem0).wait()
            pltpu.make_async_copy(us.at[srcv0], qa0, sem0).wait()
            pltpu.make_async_copy(us.at[srcv0], qb, sem0).wait()
            _compute_chunk(G2, nheads, srcv0, rowsA0, rowsB0, qa0, qb,
                           outr, wb, accsh)
            return 0

        lax.fori_loop(0, E2W // C, chunk_b, 0)

        plsc.subcore_barrier()

        for k in range((NSTRIPES + 15) // 16):
            stripe = sid + k * 16

            @pl.when(stripe < NSTRIPES)
            def _():
                pltpu.sync_copy(accsh.at[pl.ds(stripe * C, C)],
                                acc.at[cid, pl.ds(stripe * C, C)])

    scratch = [
        pltpu.VMEM((C,), _i32), pltpu.VMEM((C,), _i32),
        pltpu.VMEM((C,), _i32), pltpu.VMEM((C,), _i32),
        pltpu.VMEM((C,), _i32), pltpu.VMEM((C,), _i32),
        pltpu.VMEM((C, ROWW), _f32), pltpu.VMEM((C, ROWW), _f32),
        pltpu.VMEM((C, ROWW), _f32), pltpu.VMEM((C, ROWW), _f32),
        pltpu.VMEM((C, ROWW), _f32), pltpu.VMEM((C, ROWW), _f32),
        pltpu.VMEM((C, ROWW), _f32),
        pltpu.VMEM((C, ROWW), _f32),
    ]
    scratch += [pltpu.VMEM((C,), _f32)] * nheads
    scratch += [pltpu.VMEM_SHARED((SN, ROWW), _f32),
                pltpu.SemaphoreType.DMA, pltpu.SemaphoreType.DMA]

    mesh = plsc.VectorSubcoreMesh(core_axis_name="c", subcore_axis_name="s",
                                  num_cores=2, num_subcores=16)
    return pl.kernel(
        body,
        out_type=jax.ShapeDtypeStruct((2, SN, ROWW), _f32),
        mesh=mesh,
        scratch_types=scratch,
        compiler_params=pltpu.CompilerParams(needs_layout_passes=False,
                                             use_tc_tiling_on_sc=False),
    )


def kernel(Corpus_, batch_inputs, entity_embeddings, relation_embed, edge_list,
           edge_type, edge_embed, edge_list_nhop, edge_type_nhop, a_head0,
           a2_head0, a_head1, a2_head1, a_out, a2_out, W):
    x0p = jnp.pad(entity_embeddings, ((0, SN - N_NODES), (0, 0)))
    relp = jnp.pad(relation_embed, ((0, RP - NRELA), (0, 0)))
    eep = jnp.pad(edge_embed, ((0, E1P - E1), (0, 0)))

    srcA = jnp.concatenate([edge_list[0], jnp.full((E1P - E1,), N_NODES, _i32)]).astype(_i32)
    dstA = jnp.concatenate([edge_list[1], jnp.zeros((E1P - E1,), _i32)]).astype(_i32)
    tyA = jnp.concatenate([edge_type, jnp.zeros((E1P - E1,), _i32)]).astype(_i32)
    srcB = jnp.concatenate([edge_list_nhop[0], jnp.full((E2P - E2,), N_NODES, _i32)]).astype(_i32)
    dstB = jnp.concatenate([edge_list_nhop[1], jnp.zeros((E2P - E2,), _i32)]).astype(_i32)
    t0B = jnp.concatenate([edge_type_nhop[:, 0], jnp.zeros((E2P - E2,), _i32)]).astype(_i32)
    t1B = jnp.concatenate([edge_type_nhop[:, 1], jnp.zeros((E2P - E2,), _i32)]).astype(_i32)

    usrc, udst, t3q, rfull, r2q, ortho = pl.pallas_call(
        _tc_a1,
        out_shape=[
            jax.ShapeDtypeStruct((SN, ROWW), _f32),
            jax.ShapeDtypeStruct((SN, ROWW), _f32),
            jax.ShapeDtypeStruct((RP, ROWW), _f32),
            jax.ShapeDtypeStruct((RP, 64), _f32),
            jax.ShapeDtypeStruct((RP, ROWW), _f32),
            jax.ShapeDtypeStruct((1, 1), _f32),
        ],
    )(x0p, relp, a_head0, a_head1, a2_head0, a2_head1, a_out, a2_out, W)

    BLK = 2048
    nblk = E1P // BLK
    gq1 = pl.pallas_call(
        _tc_a2,
        grid=(nblk,),
        in_specs=[
            pl.BlockSpec((BLK, RELDIM), lambda i: (i, 0)),
            pl.BlockSpec((1, NHID), lambda i: (0, 0)),
            pl.BlockSpec((1, NHID), lambda i: (0, 0)),
            pl.BlockSpec((NHID, 2 * NFEAT + RELDIM), lambda i: (0, 0)),
            pl.BlockSpec((NHID, 2 * NFEAT + RELDIM), lambda i: (0, 0)),
        ],
        out_specs=pl.BlockSpec((BLK, ROWW), lambda i: (i, 0)),
        out_shape=jax.ShapeDtypeStruct((E1P, ROWW), _f32),
    )(eep, a2_head0, a2_head1, a_head0, a_head1)

    sc1 = _make_sc_kernel(2, LINEAR)
    acc1 = sc1(srcA, dstA, gq1, srcB, dstB, t0B, t1B, t3q, usrc, udst)

    vsrc, vdst = pl.pallas_call(
        _tc_b,
        out_shape=[
            jax.ShapeDtypeStruct((SN, ROWW), _f32),
            jax.ShapeDtypeStruct((SN, ROWW), _f32),
        ],
    )(acc1, a_out, a2_out)

    sc2 = _make_sc_kernel(1, G1)
    acc2 = sc2(srcA, dstA, tyA, srcB, dstB, t0B, t1B, r2q, vsrc, vdst)

    x = pl.pallas_call(
        _tc_c,
        out_shape=jax.ShapeDtypeStruct((N_NODES, 64), _f32),
    )(acc2)

    return (x, rfull[:NRELA], ortho[0, 0])


# speedup vs baseline: 4.2712x; 1.0231x over previous
import jax
import jax.numpy as jnp
from jax import lax
from jax.experimental import pallas as pl
from jax.experimental.pallas import tpu as pltpu
from jax.experimental.pallas import tpu_sc as plsc

N_NODES = 10000
SN = 10112
NFEAT = 128
NHID = 32
RELDIM = 64
NRELA = 500
RP = 512
ALPHA = 0.2

E1 = 160000
E1P = 163840
E1W = E1P // 32
E2 = 50000
E2P = 53248
E2W = E2P // 32
C = 128
ROWW = 72
NSTRIPES = SN // C

_f32 = jnp.float32
_i32 = jnp.int32


def _tc_a1(x0p, relp, ah0, ah1, a2h0, a2h1, aout, a2out, W,
           usrc, udst, t3q, rfull, r2q, ortho):
    A1 = jnp.concatenate([ah0[:, :NFEAT], ah1[:, :NFEAT]], axis=0)
    A2 = jnp.concatenate([ah0[:, NFEAT:2 * NFEAT], ah1[:, NFEAT:2 * NFEAT]], axis=0)
    A3 = jnp.concatenate([ah0[:, 2 * NFEAT:], ah1[:, 2 * NFEAT:]], axis=0)

    dn = (((1,), (1,)), ((), ()))

    def table2(rows, out_ref):
        s0 = lax.dot_general(rows[:, :NHID], a2h0[...], (((1,), (1,)), ((), ())))
        s1 = lax.dot_general(rows[:, NHID:], a2h1[...], (((1,), (1,)), ((), ())))
        n = rows.shape[0]
        pad = jnp.zeros((n, ROWW - 66), _f32)
        out_ref[...] = jnp.concatenate([rows, s0, s1, pad], axis=1)

    Us = lax.dot_general(x0p[...], A1, dn)
    Ud = lax.dot_general(x0p[...], A2, dn)
    table2(Us, usrc)
    table2(Ud, udst)

    T3 = lax.dot_general(relp[...], A3, dn)
    table2(T3, t3q)

    R = lax.dot_general(relp[...], W[...], (((1,), (0,)), ((), ())))
    rfull[...] = R
    B3 = aout[:, 2 * RELDIM:]
    R2 = lax.dot_general(R, B3, dn)
    sO = lax.dot_general(R2, a2out[...], (((1,), (1,)), ((), ())))
    r2q[...] = jnp.concatenate(
        [R2, sO, jnp.zeros((RP, ROWW - 65), _f32)], axis=1)

    tot = jnp.float32(0.0)
    for a in (ah0[...], ah1[...], aout[...]):
        hd = a.shape[0] // 2
        ahh = a.reshape(2, hd, a.shape[1])
        gram = lax.dot_general(ahh, ahh, (((2,), (2,)), ((0,), (0,))))
        ii = lax.broadcasted_iota(_i32, (hd, hd), 0)
        jj = lax.broadcasted_iota(_i32, (hd, hd), 1)
        eye = jnp.where(ii == jj, jnp.float32(1.0), jnp.float32(0.0))
        tot = tot + 0.01 * jnp.sum((gram - eye[None]) ** 2)
    ortho[...] = jnp.reshape(tot, (1, 1))


def _tc_a2(eeb, a2h0, a2h1, ah0, ah1, gq):
    A3 = jnp.concatenate([ah0[:, 2 * NFEAT:], ah1[:, 2 * NFEAT:]], axis=0)
    dn = (((1,), (1,)), ((), ()))
    G = lax.dot_general(eeb[...], A3, dn)
    s0 = lax.dot_general(G[:, :NHID], a2h0[...], dn)
    s1 = lax.dot_general(G[:, NHID:], a2h1[...], dn)
    pad = jnp.zeros((G.shape[0], ROWW - 66), _f32)
    gq[...] = jnp.concatenate([G, s0, s1, pad], axis=1)


def _tc_b(acc, aout, a2out, vsrc, vdst):
    s = acc[0] + acc[1]
    w0 = s[:, 64:65]
    w1 = s[:, 65:66]
    w0 = jnp.where(w0 == 0.0, jnp.float32(1e-12), w0)
    w1 = jnp.where(w1 == 0.0, jnp.float32(1e-12), w1)
    h0 = s[:, :NHID] / w0
    h1 = s[:, NHID:2 * NHID] / w1
    x1 = jnp.concatenate([_elu(h0), _elu(h1)], axis=1)
    dn = (((1,), (1,)), ((), ()))
    B1 = aout[:, :RELDIM]
    B2 = aout[:, RELDIM:2 * RELDIM]
    pad = jnp.zeros((SN, ROWW - 65), _f32)
    Vs = lax.dot_general(x1, B1, dn)
    Vd = lax.dot_general(x1, B2, dn)
    vsrc[...] = jnp.concatenate(
        [Vs, lax.dot_general(Vs, a2out[...], dn), pad], axis=1)
    vdst[...] = jnp.concatenate(
        [Vd, lax.dot_general(Vd, a2out[...], dn), pad], axis=1)


def _elu(x):
    return jnp.where(x > 0, x, jnp.exp(jnp.minimum(x, 0.0)) - 1.0)


def _tc_c(acc, out):
    s = acc[0] + acc[1]
    w = s[:, 64:65]
    w = jnp.where(w == 0.0, jnp.float32(1e-12), w)
    out[...] = (s[:, :RELDIM] / w)[:N_NODES, :]


LINEAR, G1, G2 = 0, 1, 2


def _compute_chunk(mode, nheads, srcv, rowsA, rowsB, qa, qb, outr, wb, accsh):
    seg = 64 // nheads
    lanes = lax.broadcasted_iota(_i32, (16,), 0)
    for g in range(C // 16):
        e16 = g * 16 + lanes
        for h in range(nheads):
            c16 = jnp.full((16,), 64 + h, _i32)
            sc = plsc.load_gather(rowsA, [e16, c16]) \
                + plsc.load_gather(rowsB, [e16, c16]) \
                + plsc.load_gather(qa, [e16, c16])
            if mode == G2:
                sc = sc + plsc.load_gather(qb, [e16, c16])
            w = jnp.exp(jnp.where(sc > 0, -sc, (-ALPHA) * sc))
            wb[h][pl.ds(g * 16, 16)] = w
            plsc.store_scatter(outr, [e16, c16], w)

    def edge_body(i4, _):
        for u in range(4):
            i = i4 * 4 + u
            for h in range(nheads):
                wv = plsc.load_gather(wb[h], [jnp.full((16,), i, _i32)])
                for jj in range(seg // 16):
                    j = h * (seg // 16) + jj
                    m = rowsA[i, pl.ds(j * 16, 16)] + rowsB[i, pl.ds(j * 16, 16)] \
                        + qa[i, pl.ds(j * 16, 16)]
                    if mode == G2:
                        m = m + qb[i, pl.ds(j * 16, 16)]
                    outr[i, pl.ds(j * 16, 16)] = wv * m
        return 0

    lax.fori_loop(0, C // 4, edge_body, 0)
    pltpu.sync_copy(outr, accsh.at[srcv], add=True)


def _make_sc_kernel(nheads, qmode_a):
    def body(*refs):
        if qmode_a == LINEAR:
            (srcA, dstA, gq, srcB, dstB, t0B, t1B, tq, us, ud, acc) = refs[:11]
            scr = refs[11:]
            tyA = None
        else:
            (srcA, dstA, tyA, srcB, dstB, t0B, t1B, tq, us, ud, acc) = refs[:11]
            scr = refs[11:]
            gq = None
        (srcv0, dstv0, srcv1, dstv1, t0v, t1v,
         rowsA0, rowsB0, rowsA1, rowsB1, qa0, qa1, qb, outr) = scr[:14]
        scr = scr[14:]
        wb = list(scr[:nheads]); scr = scr[nheads:]
        accsh, sem0, sem1 = scr

        cid = lax.axis_index("c")
        sid = lax.axis_index("s")
        wid = cid * 16 + sid

        z16 = jnp.zeros((16,), _f32)

        def zero_body(i, _):
            for j in range(4):
                outr[i, pl.ds(j * 16, 16)] = z16
            outr[i, pl.ds(ROWW - 16, 16)] = z16
            return 0

        lax.fori_loop(0, C, zero_body, 0)

        for k in range((NSTRIPES + 15) // 16):
            stripe = sid + k * 16

            @pl.when(stripe < NSTRIPES)
            def _():
                pltpu.sync_copy(outr, accsh.at[pl.ds(stripe * C, C)])

        plsc.subcore_barrier()

        bufs = [
            (srcv0, dstv0, rowsA0, rowsB0, qa0, sem0),
            (srcv1, dstv1, rowsA1, rowsB1, qa1, sem1),
        ]

        def issue(mode, src_h, dst_h, gq_h, t0_h, wbase, k, b):
            srcv, dstv, rowsA, rowsB, qa, sem = bufs[b]
            base = pl.multiple_of(wbase + k * C, C)
            pltpu.sync_copy(src_h.at[pl.ds(base, C)], srcv)
            pltpu.sync_copy(dst_h.at[pl.ds(base, C)], dstv)
            pltpu.async_copy(us.at[srcv], rowsA, sem)
            pltpu.async_copy(ud.at[dstv], rowsB, sem)
            if mode == LINEAR:
                pltpu.async_copy(gq_h.at[pl.ds(base, C)], qa, sem)
            else:
                tv = t0v if b == 0 else t1v
                pltpu.sync_copy(t0_h.at[pl.ds(base, C)], tv)
                pltpu.async_copy(tq.at[tv], qa, sem)

        def wait_bufs(mode, b):
            srcv, dstv, rowsA, rowsB, qa, sem = bufs[b]
            pltpu.make_async_copy(us.at[srcv], rowsA, sem).wait()
            pltpu.make_async_copy(ud.at[dstv], rowsB, sem).wait()
            pltpu.make_async_copy(us.at[srcv], qa, sem).wait()

        def consume(mode, b):
            srcv, dstv, rowsA, rowsB, qa, sem = bufs[b]
            _compute_chunk(mode, nheads, srcv, rowsA, rowsB, qa, qb,
                           outr, wb, accsh)

        nch_a = E1W // C
        assert nch_a % 2 == 0
        ia = lambda k, b: issue(qmode_a, srcA, dstA, gq, tyA, wid * E1W, k, b)
        ia(0, 0)

        def pair_body(p, _):
            k = p * 2
            ia(k + 1, 1)
            wait_bufs(qmode_a, 0)
            consume(qmode_a, 0)

            @pl.when(k + 2 < nch_a)
            def _():
                ia(k + 2, 0)

            wait_bufs(qmode_a, 1)
            consume(qmode_a, 1)
            return 0

        lax.fori_loop(0, nch_a // 2, pair_body, 0)

        def chunk_b(k, _):
            base = pl.multiple_of(wid * E2W + k * C, C)
            pltpu.sync_copy(srcB.at[pl.ds(base, C)], srcv0)
            pltpu.sync_copy(dstB.at[pl.ds(base, C)], dstv0)
            pltpu.sync_copy(t0B.at[pl.ds(base, C)], t0v)
            pltpu.sync_copy(t1B.at[pl.ds(base, C)], t1v)
            pltpu.async_copy(us.at[srcv0], rowsA0, sem0)
            pltpu.async_copy(ud.at[dstv0], rowsB0, sem0)
            pltpu.async_copy(tq.at[t0v], qa0, sem0)
            pltpu.async_copy(tq.at[t1v], qb, sem0)
            pltpu.make_async_copy(us.at[srcv0], rowsA0, sem0).wait()
            pltpu.make_async_copy(ud.at[dstv0], rowsB0, sem0).wait()
            pltpu.make_async_copy(us.at[srcv0], qa0, sem0).wait()
            pltpu.make_async_copy(us.at[srcv0], qb, sem0).wait()
            _compute_chunk(G2, nheads, srcv0, rowsA0, rowsB0, qa0, qb,
                           outr, wb, accsh)
            return 0

        lax.fori_loop(0, E2W // C, chunk_b, 0)

        plsc.subcore_barrier()

        for k in range((NSTRIPES + 15) // 16):
            stripe = sid + k * 16

            @pl.when(stripe < NSTRIPES)
            def _():
                pltpu.sync_copy(accsh.at[pl.ds(stripe * C, C)],
                                acc.at[cid, pl.ds(stripe * C, C)])

    scratch = [
        pltpu.VMEM((C,), _i32), pltpu.VMEM((C,), _i32),
        pltpu.VMEM((C,), _i32), pltpu.VMEM((C,), _i32),
        pltpu.VMEM((C,), _i32), pltpu.VMEM((C,), _i32),
        pltpu.VMEM((C, ROWW), _f32), pltpu.VMEM((C, ROWW), _f32),
        pltpu.VMEM((C, ROWW), _f32), pltpu.VMEM((C, ROWW), _f32),
        pltpu.VMEM((C, ROWW), _f32), pltpu.VMEM((C, ROWW), _f32),
        pltpu.VMEM((C, ROWW), _f32),
        pltpu.VMEM((C, ROWW), _f32),
    ]
    scratch += [pltpu.VMEM((C,), _f32)] * nheads
    scratch += [pltpu.VMEM_SHARED((SN, ROWW), _f32),
                pltpu.SemaphoreType.DMA, pltpu.SemaphoreType.DMA]

    mesh = plsc.VectorSubcoreMesh(core_axis_name="c", subcore_axis_name="s",
                                  num_cores=2, num_subcores=16)
    return pl.kernel(
        body,
        out_type=jax.ShapeDtypeStruct((2, SN, ROWW), _f32),
        mesh=mesh,
        scratch_types=scratch,
        compiler_params=pltpu.CompilerParams(needs_layout_passes=False,
                                             use_tc_tiling_on_sc=False),
    )


def kernel(Corpus_, batch_inputs, entity_embeddings, relation_embed, edge_list,
           edge_type, edge_embed, edge_list_nhop, edge_type_nhop, a_head0,
           a2_head0, a_head1, a2_head1, a_out, a2_out, W):
    x0p = jnp.pad(entity_embeddings, ((0, SN - N_NODES), (0, 0)))
    relp = jnp.pad(relation_embed, ((0, RP - NRELA), (0, 0)))
    eep = jnp.pad(edge_embed, ((0, E1P - E1), (0, 0)))

    srcA = jnp.concatenate([edge_list[0], jnp.full((E1P - E1,), N_NODES, _i32)]).astype(_i32)
    dstA = jnp.concatenate([edge_list[1], jnp.zeros((E1P - E1,), _i32)]).astype(_i32)
    tyA = jnp.concatenate([edge_type, jnp.zeros((E1P - E1,), _i32)]).astype(_i32)
    srcB = jnp.concatenate([edge_list_nhop[0], jnp.full((E2P - E2,), N_NODES, _i32)]).astype(_i32)
    dstB = jnp.concatenate([edge_list_nhop[1], jnp.zeros((E2P - E2,), _i32)]).astype(_i32)
    t0B = jnp.concatenate([edge_type_nhop[:, 0], jnp.zeros((E2P - E2,), _i32)]).astype(_i32)
    t1B = jnp.concatenate([edge_type_nhop[:, 1], jnp.zeros((E2P - E2,), _i32)]).astype(_i32)

    usrc, udst, t3q, rfull, r2q, ortho = pl.pallas_call(
        _tc_a1,
        out_shape=[
            jax.ShapeDtypeStruct((SN, ROWW), _f32),
            jax.ShapeDtypeStruct((SN, ROWW), _f32),
            jax.ShapeDtypeStruct((RP, ROWW), _f32),
            jax.ShapeDtypeStruct((RP, 64), _f32),
            jax.ShapeDtypeStruct((RP, ROWW), _f32),
            jax.ShapeDtypeStruct((1, 1), _f32),
        ],
    )(x0p, relp, a_head0, a_head1, a2_head0, a2_head1, a_out, a2_out, W)

    BLK = 2048
    nblk = E1P // BLK
    gq1 = pl.pallas_call(
        _tc_a2,
        grid=(nblk,),
        in_specs=[
            pl.BlockSpec((BLK, RELDIM), lambda i: (i, 0)),
            pl.BlockSpec((1, NHID), lambda i: (0, 0)),
            pl.BlockSpec((1, NHID), lambda i: (0, 0)),
            pl.BlockSpec((NHID, 2 * NFEAT + RELDIM), lambda i: (0, 0)),
            pl.BlockSpec((NHID, 2 * NFEAT + RELDIM), lambda i: (0, 0)),
        ],
        out_specs=pl.BlockSpec((BLK, ROWW), lambda i: (i, 0)),
        out_shape=jax.ShapeDtypeStruct((E1P, ROWW), _f32),
    )(eep, a2_head0, a2_head1, a_head0, a_head1)

    sc1 = _make_sc_kernel(2, LINEAR)
    acc1 = sc1(srcA, dstA, gq1, srcB, dstB, t0B, t1B, t3q, usrc, udst)

    vsrc, vdst = pl.pallas_call(
        _tc_b,
        out_shape=[
            jax.ShapeDtypeStruct((SN, ROWW), _f32),
            jax.ShapeDtypeStruct((SN, ROWW), _f32),
        ],
    )(acc1, a_out, a2_out)

    sc2 = _make_sc_kernel(1, G1)
    acc2 = sc2(srcA, dstA, tyA, srcB, dstB, t0B, t1B, r2q, vsrc, vdst)

    x = pl.pallas_call(
        _tc_c,
        out_shape=jax.ShapeDtypeStruct((N_NODES, 64), _f32),
    )(acc2)

    return (x, rfull[:NRELA], ortho[0, 0])
